# Initial kernel scaffold; baseline (speedup 1.0000x reference)
import jax
import jax.numpy as jnp
from jax.experimental import pallas as pl


def kernel(coordinates, numatoms, atom_index, shifts, species, rs1, inta1, params):
    raise NotImplementedError("write your pallas kernel here")


# speedup vs baseline: 6.4696x; 6.4696x over previous
import jax
import jax.numpy as jnp
from jax.experimental import pallas as pl
from jax.experimental.pallas import tpu as pltpu

_NIPSIN = 3
_CUTOFF = 6.0


def _pick_block(n, target):
    for b in range(min(n, target), 0, -1):
        if n % b == 0:
            return b
    return n


def _accum_body(nele, B):
    def body(idx0_ref, idx1_ref, shf_ref, coords_ref, rs_ref, inta_ref,
             acc_ref, g0, g1, pay):
        @pl.when(pl.program_id(0) == 0)
        def _init():
            acc_ref[...] = jnp.zeros_like(acc_ref)

        def gath(i, carry):
            a = idx0_ref[0, i]
            b = idx1_ref[0, i]
            g0[pl.ds(i, 1), :] = coords_ref[pl.ds(a, 1), :]
            g1[pl.ds(i, 1), :] = coords_ref[pl.ds(b, 1), :]
            return carry

        jax.lax.fori_loop(0, B, gath, 0)

        G0 = g0[...]
        G1 = g1[...]
        dv = G0[:, 0:3] - G1[:, 0:3] + shf_ref[...]
        d2 = jnp.sum(dv * dv, axis=1, keepdims=True)
        d = jnp.sqrt(d2)
        u = dv / d
        spec = G0[:, 3:4]

        t = jnp.minimum(d * (1.0 / _CUTOFF), 1.0)
        fc = 0.5 * (jnp.cos(jnp.pi * t) + 1.0)

        nparam = rs_ref.shape[1]
        rsB = jnp.zeros((B, nparam), dtype=jnp.float32)
        inB = jnp.zeros((B, nparam), dtype=jnp.float32)
        for s in range(nele):
            m = jnp.where(spec == float(s), 1.0, 0.0)
            rsB = rsB + m * rs_ref[s, :][None, :]
            inB = inB + m * inta_ref[s, :][None, :]

        gauss = jnp.exp(-10.0 * inB * jnp.square(d - rsB)) * fc

        ux = u[:, 0:1]
        uy = u[:, 1:2]
        uz = u[:, 2:3]
        chans = [
            gauss,
            ux * gauss, uy * gauss, uz * gauss,
            ux * ux * gauss, ux * uy * gauss, ux * uz * gauss,
            uy * uy * gauss, uy * uz * gauss, uz * uz * gauss,
        ]
        pay[...] = jnp.concatenate(chans, axis=1)

        def scat(i, carry):
            a = idx0_ref[0, i]
            acc_ref[pl.ds(a, 1), :] = (
                acc_ref[pl.ds(a, 1), :] + pay[pl.ds(i, 1), :])
            return carry

        jax.lax.fori_loop(0, B, scat, 0)

    return body


def _finish_body(nele, nparam):
    def body(T_ref, sp_ref, o_ref):
        t2 = jnp.square(T_ref[...])
        np_ = nparam
        rad = t2[:, 0:np_]
        o1 = (t2[:, 1 * np_:2 * np_] + t2[:, 2 * np_:3 * np_]
              + t2[:, 3 * np_:4 * np_])
        o2 = (t2[:, 4 * np_:5 * np_] + t2[:, 7 * np_:8 * np_]
              + t2[:, 9 * np_:10 * np_]
              + 2.0 * (t2[:, 5 * np_:6 * np_] + t2[:, 6 * np_:7 * np_]
                       + t2[:, 8 * np_:9 * np_]))
        vals = jnp.concatenate([rad, o1, o2], axis=1)
        sp = sp_ref[...]
        for s in range(nele):
            o_ref[:, s, :] = jnp.where(sp == s, vals, 0.0)

    return body


def kernel(coordinates, numatoms, atom_index, shifts, species, rs1, inta1,
           params):
    NB, NA = coordinates.shape[0], coordinates.shape[1]
    P = atom_index.shape[2]
    nele, nparam = rs1.shape

    ai = jnp.transpose(atom_index, (1, 0, 2)).reshape(2, -1)
    self_mol = jnp.repeat(jnp.arange(NB, dtype=ai.dtype) * NA, P)[None, :]
    ai12 = ai + self_mol
    idx0 = ai12[0].astype(jnp.int32)[None, :]
    idx1 = ai12[1].astype(jnp.int32)[None, :]

    coords_ = coordinates.reshape(-1, 3)
    TOT = coords_.shape[0]
    coords4 = jnp.concatenate(
        [coords_, species.astype(jnp.float32)[:, None]], axis=1)
    shifts_ = shifts.reshape(-1, 3)
    N = idx0.shape[1]

    B = N
    for cand in range(1600 - 1600 % 128, 0, -128):
        if N % cand == 0:
            B = cand
            break
    G = N // B
    nch = 10 * nparam

    acc = pl.pallas_call(
        _accum_body(nele, B),
        grid=(G,),
        in_specs=[
            pl.BlockSpec((1, B), lambda i: (0, i), memory_space=pltpu.SMEM),
            pl.BlockSpec((1, B), lambda i: (0, i), memory_space=pltpu.SMEM),
            pl.BlockSpec((B, 3), lambda i: (i, 0)),
            pl.BlockSpec((TOT, 4), lambda i: (0, 0)),
            pl.BlockSpec((nele, nparam), lambda i: (0, 0)),
            pl.BlockSpec((nele, nparam), lambda i: (0, 0)),
        ],
        out_specs=pl.BlockSpec((TOT, nch), lambda i: (0, 0)),
        out_shape=jax.ShapeDtypeStruct((TOT, nch), jnp.float32),
        scratch_shapes=[
            pltpu.VMEM((B, 4), jnp.float32),
            pltpu.VMEM((B, 4), jnp.float32),
            pltpu.VMEM((B, nch), jnp.float32),
        ],
        compiler_params=pltpu.CompilerParams(
            vmem_limit_bytes=100 * 1024 * 1024),
    )(idx0, idx1, shifts_, coords4, rs1, inta1)

    Ba = _pick_block(TOT, 1000)
    out = pl.pallas_call(
        _finish_body(nele, nparam),
        grid=(TOT // Ba,),
        in_specs=[
            pl.BlockSpec((Ba, nch), lambda i: (i, 0)),
            pl.BlockSpec((Ba, 1), lambda i: (i, 0)),
        ],
        out_specs=pl.BlockSpec((Ba, nele, 3 * nparam), lambda i: (i, 0, 0)),
        out_shape=jax.ShapeDtypeStruct((TOT, nele, 3 * nparam), jnp.float32),
    )(acc, species.astype(jnp.int32)[:, None])

    return out


# speedup vs baseline: 11.1690x; 1.7264x over previous
import jax
import jax.numpy as jnp
from jax.experimental import pallas as pl
from jax.experimental.pallas import tpu as pltpu

_NIPSIN = 3
_CUTOFF = 6.0


def _pick_block(n, target):
    for b in range(min(n, target), 0, -1):
        if n % b == 0:
            return b
    return n


def _accum_body(nele, B):
    def body(idx0_ref, idx1_ref, shf_ref, coords_ref, rs_ref, inta_ref,
             acc_ref, g0, g1, pay):
        @pl.when(pl.program_id(0) == 0)
        def _init():
            acc_ref[...] = jnp.zeros_like(acc_ref)

        def gath(i, carry):
            a = idx0_ref[0, i]
            b = idx1_ref[0, i]
            g0[pl.ds(i, 1), :] = coords_ref[pl.ds(a, 1), :]
            g1[pl.ds(i, 1), :] = coords_ref[pl.ds(b, 1), :]
            return carry

        jax.lax.fori_loop(0, B, gath, 0, unroll=8)

        G0 = g0[...]
        G1 = g1[...]
        dv = G0[:, 0:3] - G1[:, 0:3] + shf_ref[...]
        d2 = jnp.sum(dv * dv, axis=1, keepdims=True)
        d = jnp.sqrt(d2)
        u = dv / d
        spec = G0[:, 3:4]

        t = jnp.minimum(d * (1.0 / _CUTOFF), 1.0)
        fc = 0.5 * (jnp.cos(jnp.pi * t) + 1.0)

        nparam = rs_ref.shape[1]
        rsB = jnp.zeros((B, nparam), dtype=jnp.float32)
        inB = jnp.zeros((B, nparam), dtype=jnp.float32)
        for s in range(nele):
            m = jnp.where(spec == float(s), 1.0, 0.0)
            rsB = rsB + m * rs_ref[s, :][None, :]
            inB = inB + m * inta_ref[s, :][None, :]

        gauss = jnp.exp(-10.0 * inB * jnp.square(d - rsB)) * fc

        ux = u[:, 0:1]
        uy = u[:, 1:2]
        uz = u[:, 2:3]
        chans = [
            gauss,
            ux * gauss, uy * gauss, uz * gauss,
            ux * ux * gauss, ux * uy * gauss, ux * uz * gauss,
            uy * uy * gauss, uy * uz * gauss, uz * uz * gauss,
        ]
        pay[...] = jnp.concatenate(chans, axis=1)

        def scat(i, carry):
            a = idx0_ref[0, i]
            acc_ref[pl.ds(a, 1), :] = (
                acc_ref[pl.ds(a, 1), :] + pay[pl.ds(i, 1), :])
            return carry

        jax.lax.fori_loop(0, B, scat, 0, unroll=8)

    return body


def _finish_body(nele, nparam):
    def body(T_ref, sp_ref, o_ref):
        t2 = jnp.square(T_ref[...])
        np_ = nparam
        rad = t2[:, 0:np_]
        o1 = (t2[:, 1 * np_:2 * np_] + t2[:, 2 * np_:3 * np_]
              + t2[:, 3 * np_:4 * np_])
        o2 = (t2[:, 4 * np_:5 * np_] + t2[:, 7 * np_:8 * np_]
              + t2[:, 9 * np_:10 * np_]
              + 2.0 * (t2[:, 5 * np_:6 * np_] + t2[:, 6 * np_:7 * np_]
                       + t2[:, 8 * np_:9 * np_]))
        vals = jnp.concatenate([rad, o1, o2], axis=1)
        sp = sp_ref[...]
        for s in range(nele):
            o_ref[:, s, :] = jnp.where(sp == s, vals, 0.0)

    return body


def kernel(coordinates, numatoms, atom_index, shifts, species, rs1, inta1,
           params):
    NB, NA = coordinates.shape[0], coordinates.shape[1]
    P = atom_index.shape[2]
    nele, nparam = rs1.shape

    ai = jnp.transpose(atom_index, (1, 0, 2)).reshape(2, -1)
    self_mol = jnp.repeat(jnp.arange(NB, dtype=ai.dtype) * NA, P)[None, :]
    ai12 = ai + self_mol
    idx0 = ai12[0].astype(jnp.int32)[None, :]
    idx1 = ai12[1].astype(jnp.int32)[None, :]

    coords_ = coordinates.reshape(-1, 3)
    TOT = coords_.shape[0]
    coords4 = jnp.concatenate(
        [coords_, species.astype(jnp.float32)[:, None]], axis=1)
    shifts_ = shifts.reshape(-1, 3)
    N = idx0.shape[1]

    B = N
    for cand in range(1600 - 1600 % 128, 0, -128):
        if N % cand == 0:
            B = cand
            break
    G = N // B
    nch = 10 * nparam

    acc = pl.pallas_call(
        _accum_body(nele, B),
        grid=(G,),
        in_specs=[
            pl.BlockSpec((1, B), lambda i: (0, i), memory_space=pltpu.SMEM),
            pl.BlockSpec((1, B), lambda i: (0, i), memory_space=pltpu.SMEM),
            pl.BlockSpec((B, 3), lambda i: (i, 0)),
            pl.BlockSpec((TOT, 4), lambda i: (0, 0)),
            pl.BlockSpec((nele, nparam), lambda i: (0, 0)),
            pl.BlockSpec((nele, nparam), lambda i: (0, 0)),
        ],
        out_specs=pl.BlockSpec((TOT, nch), lambda i: (0, 0)),
        out_shape=jax.ShapeDtypeStruct((TOT, nch), jnp.float32),
        scratch_shapes=[
            pltpu.VMEM((B, 4), jnp.float32),
            pltpu.VMEM((B, 4), jnp.float32),
            pltpu.VMEM((B, nch), jnp.float32),
        ],
        compiler_params=pltpu.CompilerParams(
            vmem_limit_bytes=100 * 1024 * 1024),
    )(idx0, idx1, shifts_, coords4, rs1, inta1)

    Ba = _pick_block(TOT, 1000)
    out = pl.pallas_call(
        _finish_body(nele, nparam),
        grid=(TOT // Ba,),
        in_specs=[
            pl.BlockSpec((Ba, nch), lambda i: (i, 0)),
            pl.BlockSpec((Ba, 1), lambda i: (i, 0)),
        ],
        out_specs=pl.BlockSpec((Ba, nele, 3 * nparam), lambda i: (i, 0, 0)),
        out_shape=jax.ShapeDtypeStruct((TOT, nele, 3 * nparam), jnp.float32),
    )(acc, species.astype(jnp.int32)[:, None])

    return out
